# baseline (speedup 1.0000x reference)
import functools

import jax
import jax.numpy as jnp
from jax import lax
from jax.experimental import pallas as pl
from jax.experimental.pallas import tpu as pltpu
from jax.experimental.pallas import tpu_sc as plsc

_N = 10000
_E = 320000
_D = 128
_B = 8

_NC = 2
_NS = 16
_NW = _NC * _NS
_C = 128
_NBLK = _E // _C
_NCHUNK = -(-_NBLK // _NW)
_NFULL = _NBLK - (_NCHUNK - 1) * _NW

_RSTRIDE = 624
_RSIZE = 640

_BLK = 400
_GRID = _N // _BLK


def _make_mp_kernel():
    mesh = plsc.VectorSubcoreMesh(core_axis_name="c", subcore_axis_name="s")

    @functools.partial(
        pl.kernel,
        out_type=jax.ShapeDtypeStruct((_NC, _N, _D), jnp.float32),
        mesh=mesh,
        scratch_types=[
            [pltpu.VMEM((_C,), jnp.int32) for _ in range(3)],
            [pltpu.VMEM((_C,), jnp.int32) for _ in range(3)],
            [pltpu.VMEM((_C,), jnp.int32) for _ in range(3)],
            [pltpu.VMEM((_C, _D), jnp.float32) for _ in range(3)],
            pltpu.VMEM_SHARED((_N, _D), jnp.float32),
            [pltpu.SemaphoreType.DMA for _ in range(3)],
            [pltpu.SemaphoreType.DMA for _ in range(3)],
            [pltpu.SemaphoreType.DMA for _ in range(3)],
        ],
    )
    def mp(h_hbm, packed_hbm, zeros_hbm, out_hbm, pk, sb, db, rows, acc,
           pksem, gsem, ssem):
        c = lax.axis_index("c")
        s = lax.axis_index("s")
        pltpu.sync_copy(zeros_hbm, acc.at[pl.ds(s * _RSTRIDE, _RSIZE)])
        plsc.subcore_barrier()

        wid = s * _NC + c
        nch = lax.select(wid < _NFULL, _NCHUNK, _NCHUNK - 1)

        def start_pk(g, k):
            pltpu.async_copy(
                packed_hbm.at[pl.ds((g * _NW + wid) * _C, _C)], pk[k],
                pksem[k])

        def unpack(k):
            for j in range(_C // 16):
                p = pk[k][pl.ds(j * 16, 16)]
                sb[k][pl.ds(j * 16, 16)] = lax.bitwise_and(p, 0xFFFF)
                db[k][pl.ds(j * 16, 16)] = lax.shift_right_logical(p, 16)

        def drain_pk(k):
            pltpu.make_async_copy(
                packed_hbm.at[pl.ds(0, _C)], pk[k], pksem[k]).wait()

        def drain_rows(k, sem):
            pltpu.make_async_copy(
                h_hbm.at[pl.ds(0, _C)], rows[k], sem[k]).wait()

        def step(pk_next, kg, kn, kp, drain_ss, do_next, do_pk):
            drain_rows(kg, gsem)
            pltpu.async_copy(rows[kg], acc.at[db[kg]], ssem[kg], add=True)
            if do_next:
                if drain_ss:
                    drain_rows(kn, ssem)
                drain_pk(kn)
                unpack(kn)
                pltpu.async_copy(h_hbm.at[sb[kn]], rows[kn], gsem[kn])
                if do_pk:
                    start_pk(pk_next, kp)

        start_pk(0, 0)
        drain_pk(0)
        unpack(0)
        pltpu.async_copy(h_hbm.at[sb[0]], rows[0], gsem[0])
        start_pk(1, 1)

        step(2, 0, 1, 2, False, True, True)
        step(3, 1, 2, 0, False, True, True)
        step(4, 2, 0, 1, True, True, True)

        def body(k, carry):
            g = 3 * k
            step(g + 2, 0, 1, 2, True, True, True)
            step(g + 3, 1, 2, 0, True, True, True)
            step(g + 4, 2, 0, 1, True, True, True)
            return carry

        lax.fori_loop(1, 25, body, 0)

        step(77, 0, 1, 2, True, True, True)
        drain_rows(1, gsem)
        pltpu.async_copy(rows[1], acc.at[db[1]], ssem[1], add=True)
        drain_rows(2, ssem)
        drain_pk(2)
        unpack(2)
        pltpu.async_copy(h_hbm.at[sb[2]], rows[2], gsem[2])

        @pl.when(nch == _NCHUNK)
        def _():
            start_pk(_NCHUNK - 1, 0)

        drain_rows(2, gsem)
        pltpu.async_copy(rows[2], acc.at[db[2]], ssem[2], add=True)

        @pl.when(nch == _NCHUNK)
        def _():
            drain_rows(0, ssem)
            drain_pk(0)
            unpack(0)
            pltpu.async_copy(h_hbm.at[sb[0]], rows[0], gsem[0])
            drain_rows(0, gsem)
            pltpu.async_copy(rows[0], acc.at[db[0]], ssem[0], add=True)

        drain_rows(0, ssem)
        drain_rows(1, ssem)
        drain_rows(2, ssem)

        plsc.subcore_barrier()
        pltpu.sync_copy(acc.at[pl.ds(s * _RSTRIDE, _RSIZE)],
                        out_hbm.at[c, pl.ds(s * _RSTRIDE, _RSIZE)])

    return mp


_mp_kernel = _make_mp_kernel()


def _mm_body(a0_ref, a1_ref, w_ref, b_ref, o_ref):
    agg = a0_ref[...] + a1_ref[...]
    h = jnp.dot(agg, w_ref[...], preferred_element_type=jnp.float32)
    o_ref[...] = jnp.maximum(h + b_ref[...], 0.0)


def _mm_relu(a0, a1, w, b):
    return pl.pallas_call(
        _mm_body,
        grid=(_GRID,),
        in_specs=[
            pl.BlockSpec((_BLK, _D), lambda i: (i, 0)),
            pl.BlockSpec((_BLK, _D), lambda i: (i, 0)),
            pl.BlockSpec((_D, _D), lambda i: (0, 0)),
            pl.BlockSpec((1, _D), lambda i: (0, 0)),
        ],
        out_specs=pl.BlockSpec((_BLK, _D), lambda i: (i, 0)),
        out_shape=jax.ShapeDtypeStruct((_N, _D), jnp.float32),
    )(a0, a1, w, b)


def _mm_pool_body(a0_ref, a1_ref, w_ref, b_ref, batch_ref, o_ref,
                  sums_ref, counts_ref):
    i = pl.program_id(0)

    @pl.when(i == 0)
    def _init():
        sums_ref[...] = jnp.zeros_like(sums_ref)
        counts_ref[...] = jnp.zeros_like(counts_ref)

    agg = a0_ref[...] + a1_ref[...]
    h = jnp.dot(agg, w_ref[...], preferred_element_type=jnp.float32)
    h = jnp.maximum(h + b_ref[...], 0.0)

    bvec = batch_ref[0, 0, :]
    onehot = (bvec[None, :] == lax.broadcasted_iota(jnp.int32, (_B, _BLK), 0)
              ).astype(jnp.float32)
    sums_ref[...] += jnp.dot(onehot, h, preferred_element_type=jnp.float32)
    counts_ref[...] += jnp.broadcast_to(
        jnp.sum(onehot, axis=1, keepdims=True), (_B, _D))

    @pl.when(i == _GRID - 1)
    def _fin():
        o_ref[...] = sums_ref[...] / jnp.maximum(counts_ref[...], 1.0)


def _mm_relu_pool(a0, a1, w, b, batch3d):
    return pl.pallas_call(
        _mm_pool_body,
        grid=(_GRID,),
        in_specs=[
            pl.BlockSpec((_BLK, _D), lambda i: (i, 0)),
            pl.BlockSpec((_BLK, _D), lambda i: (i, 0)),
            pl.BlockSpec((_D, _D), lambda i: (0, 0)),
            pl.BlockSpec((1, _D), lambda i: (0, 0)),
            pl.BlockSpec((1, 1, _BLK), lambda i: (i, 0, 0)),
        ],
        out_specs=pl.BlockSpec((_B, _D), lambda i: (0, 0)),
        out_shape=jax.ShapeDtypeStruct((_B, _D), jnp.float32),
        scratch_shapes=[
            pltpu.VMEM((_B, _D), jnp.float32),
            pltpu.VMEM((_B, _D), jnp.float32),
        ],
    )(a0, a1, w, b, batch3d)


@jax.jit
def kernel(x, edge_index, batch, W0, b0, W1, b1):
    packed = edge_index[0] + edge_index[1] * 65536
    zeros = jnp.zeros((_RSIZE, _D), dtype=jnp.float32)

    parts = _mp_kernel(x, packed, zeros)
    h1 = _mm_relu(parts[0], parts[1], W0, b0.reshape(1, _D))

    parts2 = _mp_kernel(h1, packed, zeros)
    batch3d = batch.reshape(_GRID, 1, _BLK)
    emb = _mm_relu_pool(parts2[0], parts2[1], W1, b1.reshape(1, _D), batch3d)
    return emb


# speedup vs baseline: 1.3488x; 1.3488x over previous
import functools

import jax
import jax.numpy as jnp
from jax import lax
from jax.experimental import pallas as pl
from jax.experimental.pallas import tpu as pltpu
from jax.experimental.pallas import tpu_sc as plsc

_N = 10000
_E = 320000
_D = 128
_B = 8

_NC = 2
_NS = 16
_NW = _NC * _NS
_C = 128
_NBLK = _E // _C
_NCHUNK = -(-_NBLK // _NW)
_NFULL = _NBLK - (_NCHUNK - 1) * _NW

_RSTRIDE = 624
_RSIZE = 640

_BLK = 2000
_GRID = _N // _BLK


def _make_mp_kernel():
    mesh = plsc.VectorSubcoreMesh(core_axis_name="c", subcore_axis_name="s")

    @functools.partial(
        pl.kernel,
        out_type=jax.ShapeDtypeStruct((_NC, _N, _D), jnp.float32),
        mesh=mesh,
        scratch_types=[
            [pltpu.VMEM((_C,), jnp.int32) for _ in range(3)],
            [pltpu.VMEM((_C,), jnp.int32) for _ in range(3)],
            [pltpu.VMEM((_C,), jnp.int32) for _ in range(3)],
            [pltpu.VMEM((_C, _D), jnp.float32) for _ in range(3)],
            pltpu.VMEM_SHARED((_N, _D), jnp.float32),
            [pltpu.SemaphoreType.DMA for _ in range(3)],
            [pltpu.SemaphoreType.DMA for _ in range(3)],
            [pltpu.SemaphoreType.DMA for _ in range(3)],
            pltpu.SemaphoreType.DMA,
        ],
    )
    def mp(h_hbm, packed_hbm, zeros_hbm, out_hbm, pk, sb, db, rows, acc,
           pksem, gsem, ssem, zsem):
        c = lax.axis_index("c")
        s = lax.axis_index("s")
        az = pltpu.async_copy(zeros_hbm, acc.at[pl.ds(s * _RSTRIDE, _RSIZE)],
                              zsem)

        wid = s * _NC + c
        nch = lax.select(wid < _NFULL, _NCHUNK, _NCHUNK - 1)

        def start_pk(g, k):
            pltpu.async_copy(
                packed_hbm.at[pl.ds((g * _NW + wid) * _C, _C)], pk[k],
                pksem[k])

        def unpack(k):
            for j in range(_C // 16):
                p = pk[k][pl.ds(j * 16, 16)]
                sb[k][pl.ds(j * 16, 16)] = lax.bitwise_and(p, 0xFFFF)
                db[k][pl.ds(j * 16, 16)] = lax.shift_right_logical(p, 16)

        def drain_pk(k):
            pltpu.make_async_copy(
                packed_hbm.at[pl.ds(0, _C)], pk[k], pksem[k]).wait()

        def drain_rows(k, sem):
            pltpu.make_async_copy(
                h_hbm.at[pl.ds(0, _C)], rows[k], sem[k]).wait()

        start_pk(0, 0)
        start_pk(1, 1)
        drain_pk(0)
        unpack(0)
        pltpu.async_copy(h_hbm.at[sb[0]], rows[0], gsem[0])
        az.wait()
        plsc.subcore_barrier()

        def body(g, carry):
            def step(kg, kn, kp):
                @pl.when(g + 1 < nch)
                def _():
                    @pl.when(g >= 2)
                    def _():
                        drain_rows(kn, ssem)
                    drain_pk(kn)
                    unpack(kn)
                    pltpu.async_copy(h_hbm.at[sb[kn]], rows[kn], gsem[kn])

                    @pl.when(g + 2 < nch)
                    def _():
                        start_pk(g + 2, kp)

                drain_rows(kg, gsem)
                pltpu.async_copy(rows[kg], acc.at[db[kg]], ssem[kg],
                                 add=True)

            @pl.when(g % 3 == 0)
            def _():
                step(0, 1, 2)

            @pl.when(g % 3 == 1)
            def _():
                step(1, 2, 0)

            @pl.when(g % 3 == 2)
            def _():
                step(2, 0, 1)

            return carry

        lax.fori_loop(0, nch, body, 0)

        drain_rows(0, ssem)
        drain_rows(1, ssem)
        drain_rows(2, ssem)

        plsc.subcore_barrier()
        pltpu.sync_copy(acc.at[pl.ds(s * _RSTRIDE, _RSIZE)],
                        out_hbm.at[c, pl.ds(s * _RSTRIDE, _RSIZE)])

    return mp


_mp_kernel = _make_mp_kernel()


def _mm_body(a0_ref, a1_ref, w_ref, b_ref, o_ref):
    agg = a0_ref[...] + a1_ref[...]
    h = jnp.dot(agg, w_ref[...], preferred_element_type=jnp.float32)
    o_ref[...] = jnp.maximum(h + b_ref[...], 0.0)


def _mm_relu(a0, a1, w, b):
    return pl.pallas_call(
        _mm_body,
        grid=(_GRID,),
        in_specs=[
            pl.BlockSpec((_BLK, _D), lambda i: (i, 0)),
            pl.BlockSpec((_BLK, _D), lambda i: (i, 0)),
            pl.BlockSpec((_D, _D), lambda i: (0, 0)),
            pl.BlockSpec((1, _D), lambda i: (0, 0)),
        ],
        out_specs=pl.BlockSpec((_BLK, _D), lambda i: (i, 0)),
        out_shape=jax.ShapeDtypeStruct((_N, _D), jnp.float32),
    )(a0, a1, w, b)


def _mm_pool_body(a0_ref, a1_ref, w_ref, b_ref, batch_ref, o_ref,
                  sums_ref, counts_ref):
    i = pl.program_id(0)

    @pl.when(i == 0)
    def _init():
        sums_ref[...] = jnp.zeros_like(sums_ref)
        counts_ref[...] = jnp.zeros_like(counts_ref)

    agg = a0_ref[...] + a1_ref[...]
    h = jnp.dot(agg, w_ref[...], preferred_element_type=jnp.float32)
    h = jnp.maximum(h + b_ref[...], 0.0)

    bvec = batch_ref[0, 0, :]
    onehot = (bvec[None, :] == lax.broadcasted_iota(jnp.int32, (_B, _BLK), 0)
              ).astype(jnp.float32)
    sums_ref[...] += jnp.dot(onehot, h, preferred_element_type=jnp.float32)
    counts_ref[...] += jnp.broadcast_to(
        jnp.sum(onehot, axis=1, keepdims=True), (_B, _D))

    @pl.when(i == _GRID - 1)
    def _fin():
        o_ref[...] = sums_ref[...] / jnp.maximum(counts_ref[...], 1.0)


def _mm_relu_pool(a0, a1, w, b, batch3d):
    return pl.pallas_call(
        _mm_pool_body,
        grid=(_GRID,),
        in_specs=[
            pl.BlockSpec((_BLK, _D), lambda i: (i, 0)),
            pl.BlockSpec((_BLK, _D), lambda i: (i, 0)),
            pl.BlockSpec((_D, _D), lambda i: (0, 0)),
            pl.BlockSpec((1, _D), lambda i: (0, 0)),
            pl.BlockSpec((1, 1, _BLK), lambda i: (i, 0, 0)),
        ],
        out_specs=pl.BlockSpec((_B, _D), lambda i: (0, 0)),
        out_shape=jax.ShapeDtypeStruct((_B, _D), jnp.float32),
        scratch_shapes=[
            pltpu.VMEM((_B, _D), jnp.float32),
            pltpu.VMEM((_B, _D), jnp.float32),
        ],
    )(a0, a1, w, b, batch3d)


@jax.jit
def kernel(x, edge_index, batch, W0, b0, W1, b1):
    packed = edge_index[0] + edge_index[1] * 65536
    zeros = jnp.zeros((_RSIZE, _D), dtype=jnp.float32)

    parts = _mp_kernel(x, packed, zeros)
    h1 = _mm_relu(parts[0], parts[1], W0, b0.reshape(1, _D))

    parts2 = _mp_kernel(h1, packed, zeros)
    batch3d = batch.reshape(_GRID, 1, _BLK)
    emb = _mm_relu_pool(parts2[0], parts2[1], W1, b1.reshape(1, _D), batch3d)
    return emb
